# baseline (speedup 1.0000x reference)
import functools
import math

import jax
import jax.numpy as jnp
import numpy as np
from jax import lax
from jax.experimental import pallas as pl
from jax.experimental.pallas import tpu as pltpu
from jax.experimental.pallas import tpu_sc as plsc

NUM_EMB = 100000
DIM = 32
BATCH = 16384
HIST = 200

NC = 2
NS = 16
NW = NC * NS

B_TOT = BATCH * HIST
B_PER_W = B_TOT // NW
BATCH_PER_W = BATCH // NW
CHUNKB = 8
CHUNK = CHUNKB * HIST
N_CHUNK = BATCH_PER_W // CHUNKB
assert B_PER_W * NW == B_TOT and N_CHUNK * CHUNKB == BATCH_PER_W


def _pe_host() -> np.ndarray:
    position = np.arange(0, NUM_EMB, dtype=np.float32)[:, None]
    div_term = np.exp(
        np.arange(0, DIM, 2, dtype=np.float32) * (-(math.log(10000.0) / DIM))
    )
    pe = np.zeros((NUM_EMB, DIM), dtype=np.float32)
    pe[:, 0::2] = np.sin(position * div_term)
    pe[:, 1::2] = np.cos(position * div_term)
    return pe


_PE = _pe_host()


def _add_body(w_ref, pe_ref, out_ref):
    out_ref[...] = w_ref[...] + pe_ref[...]


def _table_add(emb_weight):
    w = emb_weight.reshape(25000, 128)
    pe = jnp.asarray(_PE.reshape(25000, 128))
    out = pl.pallas_call(
        _add_body,
        out_shape=jax.ShapeDtypeStruct((25000, 128), jnp.float32),
        grid=(5,),
        in_specs=[
            pl.BlockSpec((5000, 128), lambda i: (i, 0)),
            pl.BlockSpec((5000, 128), lambda i: (i, 0)),
        ],
        out_specs=pl.BlockSpec((5000, 128), lambda i: (i, 0)),
    )(w, pe)
    return out.reshape(NUM_EMB, DIM)


NBUF = 2
assert N_CHUNK % NBUF == 0


def _gather_body(table_hbm, idx_hbm, out_hbm, idx_v, rows_v,
                 idx_s0, idx_s1, gat_s0, gat_s1, out_s0, out_s1):
    wid = lax.axis_index("s") * NC + lax.axis_index("c")
    base = wid * B_PER_W
    bbase = wid * BATCH_PER_W
    idx_sems = (idx_s0, idx_s1)
    gat_sems = (gat_s0, gat_s1)
    out_sems = (out_s0, out_s1)

    def idx_load(b, g):
        return pltpu.make_async_copy(
            idx_hbm.at[pl.ds(base + g * CHUNK, CHUNK)], idx_v.at[b], idx_sems[b])

    def gather(b):
        return pltpu.make_async_copy(
            table_hbm.at[idx_v.at[b]], rows_v.at[b], gat_sems[b])

    def writeback(b, g):
        return pltpu.make_async_copy(
            rows_v.at[b], out_hbm.at[pl.ds(base + g * CHUNK, CHUNK)], out_sems[b])

    for b in range(NBUF):
        idx_load(b, b).start()

    def outer(t, carry):
        go = t * NBUF
        for b in range(NBUF):
            g = go + b
            idx_load(b, g).wait()

            @pl.when(g >= NBUF)
            def _():
                writeback(b, 0).wait()

            gather(b).start()
            gather(b).wait()

            @pl.when(g + NBUF < N_CHUNK)
            def _():
                idx_load(b, g + NBUF).start()

            writeback(b, g).start()
        return carry

    lax.fori_loop(0, N_CHUNK // NBUF, outer, 0)

    for b in range(NBUF):
        writeback(b, 0).wait()


def _gather_sc(table, idx_flat):
    mesh = plsc.VectorSubcoreMesh(core_axis_name="c", subcore_axis_name="s")
    k = functools.partial(
        pl.kernel,
        mesh=mesh,
        out_type=jax.ShapeDtypeStruct((B_TOT, DIM), jnp.float32),
        scratch_types=[
            pltpu.VMEM((NBUF, CHUNK), jnp.int32),
            pltpu.VMEM((NBUF, CHUNK, DIM), jnp.float32),
            pltpu.SemaphoreType.DMA,
            pltpu.SemaphoreType.DMA,
            pltpu.SemaphoreType.DMA,
            pltpu.SemaphoreType.DMA,
            pltpu.SemaphoreType.DMA,
            pltpu.SemaphoreType.DMA,
        ],
        compiler_params=pltpu.CompilerParams(use_tc_tiling_on_sc=False),
    )(_gather_body)
    return k(table, idx_flat)


TB = 2048


def _tr_body(in_ref, out_ref):
    x = in_ref[...]
    xt = x.T
    for q in range(4):
        out_ref[0, :, pl.ds(q * (TB // 4), TB // 4)] = (
            xt[32 * q:32 * (q + 1), :])


def _transpose_tc(m128):
    return pl.pallas_call(
        _tr_body,
        out_shape=jax.ShapeDtypeStruct((HIST, DIM, BATCH), jnp.float32),
        grid=(HIST, BATCH // TB),
        in_specs=[pl.BlockSpec((TB // 4, 128),
                               lambda h, b: (h * (BATCH // TB) + b, 0))],
        out_specs=pl.BlockSpec((1, DIM, TB), lambda h, b: (h, 0, b)),
    )(m128)


def kernel(indices, emb_weight):
    table = _table_add(emb_weight)
    idx_flat = indices.T.reshape(B_TOT).astype(jnp.int32)
    idx_flat = (idx_flat.reshape(B_TOT // TB, 4, TB // 4)
                .swapaxes(1, 2).reshape(B_TOT))
    m = _gather_sc(table, idx_flat)
    t3 = _transpose_tc(m.reshape(B_TOT // 4, 128))
    return jnp.transpose(t3, (2, 0, 1))


# speedup vs baseline: 1.0007x; 1.0007x over previous
import functools
import math

import jax
import jax.numpy as jnp
import numpy as np
from jax import lax
from jax.experimental import pallas as pl
from jax.experimental.pallas import tpu as pltpu
from jax.experimental.pallas import tpu_sc as plsc

NUM_EMB = 100000
DIM = 32
BATCH = 16384
HIST = 200

NC = 2
NS = 16
NW = NC * NS

B_TOT = BATCH * HIST
B_PER_W = B_TOT // NW
TB = 2048
CHUNK = 1024
KSUB = CHUNK // 128
N_CHUNK = B_PER_W // CHUNK
NBUF = 2
assert B_PER_W * NW == B_TOT and N_CHUNK * CHUNK == B_PER_W
assert B_PER_W % TB == 0 and N_CHUNK % NBUF == 0


def _pe_host() -> np.ndarray:
    position = np.arange(0, NUM_EMB, dtype=np.float32)[:, None]
    div_term = np.exp(
        np.arange(0, DIM, 2, dtype=np.float32) * (-(math.log(10000.0) / DIM))
    )
    pe = np.zeros((NUM_EMB, DIM), dtype=np.float32)
    pe[:, 0::2] = np.sin(position * div_term)
    pe[:, 1::2] = np.cos(position * div_term)
    return pe


_PE = _pe_host()


def _add_body(w_ref, pe_ref, out_ref):
    out_ref[...] = w_ref[...] + pe_ref[...]


def _table_add(emb_weight):
    w = emb_weight.reshape(25000, 128)
    pe = jnp.asarray(_PE.reshape(25000, 128))
    out = pl.pallas_call(
        _add_body,
        out_shape=jax.ShapeDtypeStruct((25000, 128), jnp.float32),
        grid=(5,),
        in_specs=[
            pl.BlockSpec((5000, 128), lambda i: (i, 0)),
            pl.BlockSpec((5000, 128), lambda i: (i, 0)),
        ],
        out_specs=pl.BlockSpec((5000, 128), lambda i: (i, 0)),
    )(w, pe)
    return out.reshape(NUM_EMB, DIM)


def _gather_body(table_hbm, idx_hbm, out_hbm, idx_v, rows_v,
                 idx_s0, idx_s1, gat_s0, gat_s1, out_s0, out_s1):
    wid = lax.axis_index("s") * NC + lax.axis_index("c")
    base = wid * (B_PER_W // 128)
    idx_sems = (idx_s0, idx_s1)
    gat_sems = (gat_s0, gat_s1)
    out_sems = (out_s0, out_s1)

    def idx_load(b, g):
        return pltpu.make_async_copy(
            idx_hbm.at[pl.ds(base + g * KSUB, KSUB)], idx_v.at[b],
            idx_sems[b])

    def gathers(b):
        return [
            pltpu.make_async_copy(
                table_hbm.at[idx_v.at[b, j]], rows_v.at[b, j], gat_sems[b])
            for j in range(KSUB)
        ]

    def writeback(b, g):
        return pltpu.make_async_copy(
            rows_v.at[b], out_hbm.at[pl.ds(base + g * KSUB, KSUB)],
            out_sems[b])

    for b in range(NBUF):
        idx_load(b, b).start()

    def outer(t, carry):
        go = t * NBUF
        for b in range(NBUF):
            g = go + b
            idx_load(b, g).wait()

            @pl.when(g >= NBUF)
            def _():
                writeback(b, 0).wait()

            for c in gathers(b):
                c.start()
        for b in range(NBUF):
            g = go + b
            for c in gathers(b):
                c.wait()

            @pl.when(g + NBUF < N_CHUNK)
            def _():
                idx_load(b, g + NBUF).start()

            writeback(b, g).start()
        return carry

    lax.fori_loop(0, N_CHUNK // NBUF, outer, 0)

    for b in range(NBUF):
        writeback(b, 0).wait()


def _gather_sc(table, idx_2d):
    k = functools.partial(
        pl.kernel,
        mesh=plsc.VectorSubcoreMesh(core_axis_name="c", subcore_axis_name="s"),
        out_type=jax.ShapeDtypeStruct((B_TOT // 128, 128, DIM), jnp.float32),
        scratch_types=[
            pltpu.VMEM((NBUF, KSUB, 128), jnp.int32),
            pltpu.VMEM((NBUF, KSUB, 128, DIM), jnp.float32),
            pltpu.SemaphoreType.DMA,
            pltpu.SemaphoreType.DMA,
            pltpu.SemaphoreType.DMA,
            pltpu.SemaphoreType.DMA,
            pltpu.SemaphoreType.DMA,
            pltpu.SemaphoreType.DMA,
        ],
        compiler_params=pltpu.CompilerParams(use_tc_tiling_on_sc=False),
    )(_gather_body)
    return k(table, idx_2d)


def _tr_body(in_ref, out_ref):
    x = in_ref[...]
    xt = x.T
    for q in range(4):
        out_ref[0, :, pl.ds(q * (TB // 4), TB // 4)] = (
            xt[32 * q:32 * (q + 1), :])


def _transpose_tc(m128):
    return pl.pallas_call(
        _tr_body,
        out_shape=jax.ShapeDtypeStruct((HIST, DIM, BATCH), jnp.float32),
        grid=(HIST, BATCH // TB),
        in_specs=[pl.BlockSpec((TB // 4, 128),
                               lambda h, b: (h * (BATCH // TB) + b, 0))],
        out_specs=pl.BlockSpec((1, DIM, TB), lambda h, b: (h, 0, b)),
    )(m128)


def kernel(indices, emb_weight):
    table = _table_add(emb_weight)
    idx_flat = indices.T.reshape(B_TOT).astype(jnp.int32)
    idx_flat = (idx_flat.reshape(B_TOT // TB, 4, TB // 4)
                .swapaxes(1, 2).reshape(B_TOT))
    m = _gather_sc(table, idx_flat.reshape(B_TOT // 128, 128))
    t3 = _transpose_tc(m.reshape(B_TOT // 4, 128))
    return jnp.transpose(t3, (2, 0, 1))
